# baseline (speedup 1.0000x reference)
import functools

import jax
import jax.numpy as jnp
from jax import lax
from jax.experimental import pallas as pl
from jax.experimental.pallas import tpu as pltpu
from jax.experimental.pallas import tpu_sc as plsc

N = 10000
E = 320000
D = 128

NC = 2
NS = 16
NW = NC * NS
EPW = E // NW
CH = 128
NF = 80
EPAD = NF * CH - EPW
ACC_ROWS = N + 8

SLAB = 624
SLAB_TAIL = N - NS * SLAB

DEG_PER_TILE = 632
DEG_PAD = NS * DEG_PER_TILE

_MESH = plsc.VectorSubcoreMesh(core_axis_name="c", subcore_axis_name="s")


@functools.partial(
    pl.kernel,
    out_type=jax.ShapeDtypeStruct((NC, DEG_PAD, D), jnp.float32),
    mesh=_MESH,
    scratch_types=[
        pltpu.VMEM_SHARED((DEG_PAD, D), jnp.float32),
        pltpu.VMEM((NF, CH), jnp.int32),
        pltpu.VMEM((CH, D), jnp.float32),
    ],
)
def _sc_deg(dstm_hbm, zeros_hbm, out_hbm, acc_sh, dst2d, ones_v):
    c = lax.axis_index("c")
    s = lax.axis_index("s")
    w = c * NS + s

    def fill_ones(i, _):
        r = i // 8
        k = i % 8
        ones_v[r, pl.ds(k * 16, 16)] = jnp.ones((16,), jnp.float32)
        return 0

    lax.fori_loop(0, CH * 8, fill_ones, 0)

    pltpu.sync_copy(dstm_hbm.at[w], dst2d)

    pltpu.sync_copy(
        zeros_hbm.at[pl.ds(s * DEG_PER_TILE, DEG_PER_TILE)],
        acc_sh.at[pl.ds(s * DEG_PER_TILE, DEG_PER_TILE)],
    )
    plsc.subcore_barrier()

    def body(j, _):
        pltpu.sync_copy(ones_v, acc_sh.at[dst2d.at[j]], add=True)
        return 0

    lax.fori_loop(0, NF, body, 0)

    plsc.subcore_barrier()
    pltpu.sync_copy(
        acc_sh.at[pl.ds(s * DEG_PER_TILE, DEG_PER_TILE)],
        out_hbm.at[c, pl.ds(s * DEG_PER_TILE, DEG_PER_TILE)],
    )


@functools.partial(
    pl.kernel,
    out_type=jax.ShapeDtypeStruct((NC, N, D), jnp.float32),
    mesh=_MESH,
    scratch_types=[
        pltpu.VMEM_SHARED((ACC_ROWS, D), jnp.float32),
        pltpu.VMEM((NF, CH), jnp.int32),
        pltpu.VMEM((NF, CH), jnp.int32),
        pltpu.VMEM((CH, D), jnp.float32),
        pltpu.SemaphoreType.DMA,
    ],
)
def _sc_agg(y_hbm, srcp_hbm, dstp_hbm, zeros_hbm, out_hbm,
            acc_sh, src2d, dst2d, rows, sem):
    c = lax.axis_index("c")
    s = lax.axis_index("s")
    w = c * NS + s

    pltpu.sync_copy(srcp_hbm.at[w], src2d)
    pltpu.sync_copy(dstp_hbm.at[w], dst2d)

    pltpu.sync_copy(
        zeros_hbm.at[pl.ds(s * SLAB, SLAB)],
        acc_sh.at[pl.ds(s * SLAB, SLAB)],
    )

    @pl.when(s == 0)
    def _():
        pltpu.sync_copy(
            zeros_hbm.at[pl.ds(NS * SLAB, SLAB_TAIL)],
            acc_sh.at[pl.ds(NS * SLAB, SLAB_TAIL)],
        )

    plsc.subcore_barrier()

    def body(j, _):
        pltpu.async_copy(y_hbm.at[src2d.at[j]], rows, sem).wait()
        pltpu.sync_copy(rows, acc_sh.at[dst2d.at[j]], add=True)
        return 0

    lax.fori_loop(0, NF, body, 0)

    plsc.subcore_barrier()
    pltpu.sync_copy(
        acc_sh.at[pl.ds(s * SLAB, SLAB)],
        out_hbm.at[c, pl.ds(s * SLAB, SLAB)],
    )

    @pl.when(s == 0)
    def _():
        pltpu.sync_copy(
            acc_sh.at[pl.ds(NS * SLAB, SLAB_TAIL)],
            out_hbm.at[c, pl.ds(NS * SLAB, SLAB_TAIL)],
        )


_BLK = 2000


def _tc_pre_body(x_ref, ws_ref, wn_ref, b_ref, z_ref, y_ref):
    x = x_ref[...]
    z_ref[...] = jnp.dot(x, ws_ref[...], preferred_element_type=jnp.float32) + b_ref[...]
    y_ref[...] = jnp.dot(x, wn_ref[...], preferred_element_type=jnp.float32)


def _tc_pre(x, ws, wn, b):
    return pl.pallas_call(
        _tc_pre_body,
        grid=(N // _BLK,),
        in_specs=[
            pl.BlockSpec((_BLK, D), lambda i: (i, 0)),
            pl.BlockSpec((D, D), lambda i: (0, 0)),
            pl.BlockSpec((D, D), lambda i: (0, 0)),
            pl.BlockSpec((1, D), lambda i: (0, 0)),
        ],
        out_specs=[
            pl.BlockSpec((_BLK, D), lambda i: (i, 0)),
            pl.BlockSpec((_BLK, D), lambda i: (i, 0)),
        ],
        out_shape=[
            jax.ShapeDtypeStruct((N, D), jnp.float32),
            jax.ShapeDtypeStruct((N, D), jnp.float32),
        ],
    )(x, ws, wn, b)


def _combine(z_ref, acc_ref, deg_ref):
    d = deg_ref[0] + deg_ref[1]
    inv = 1.0 / jnp.maximum(d, 1.0)
    return z_ref[...] + (acc_ref[0] + acc_ref[1]) * inv


def _tc_postpre_body(z_ref, acc_ref, deg_ref, ws_ref, wn_ref, b_ref, z2_ref, y2_ref):
    h = jnp.maximum(_combine(z_ref, acc_ref, deg_ref), 0.0)
    z2_ref[...] = jnp.dot(h, ws_ref[...], preferred_element_type=jnp.float32) + b_ref[...]
    y2_ref[...] = jnp.dot(h, wn_ref[...], preferred_element_type=jnp.float32)


def _tc_postpre(z, acc, deg, ws, wn, b):
    return pl.pallas_call(
        _tc_postpre_body,
        grid=(N // _BLK,),
        in_specs=[
            pl.BlockSpec((_BLK, D), lambda i: (i, 0)),
            pl.BlockSpec((NC, _BLK, D), lambda i: (0, i, 0)),
            pl.BlockSpec((NC, _BLK, 1), lambda i: (0, i, 0)),
            pl.BlockSpec((D, D), lambda i: (0, 0)),
            pl.BlockSpec((D, D), lambda i: (0, 0)),
            pl.BlockSpec((1, D), lambda i: (0, 0)),
        ],
        out_specs=[
            pl.BlockSpec((_BLK, D), lambda i: (i, 0)),
            pl.BlockSpec((_BLK, D), lambda i: (i, 0)),
        ],
        out_shape=[
            jax.ShapeDtypeStruct((N, D), jnp.float32),
            jax.ShapeDtypeStruct((N, D), jnp.float32),
        ],
    )(z, acc, deg, ws, wn, b)


def _tc_post_body(z_ref, acc_ref, deg_ref, out_ref):
    out_ref[...] = _combine(z_ref, acc_ref, deg_ref)


def _tc_post(z, acc, deg):
    return pl.pallas_call(
        _tc_post_body,
        grid=(N // _BLK,),
        in_specs=[
            pl.BlockSpec((_BLK, D), lambda i: (i, 0)),
            pl.BlockSpec((NC, _BLK, D), lambda i: (0, i, 0)),
            pl.BlockSpec((NC, _BLK, 1), lambda i: (0, i, 0)),
        ],
        out_specs=pl.BlockSpec((_BLK, D), lambda i: (i, 0)),
        out_shape=jax.ShapeDtypeStruct((N, D), jnp.float32),
    )(z, acc, deg)


def kernel(inputs, Ws0, Wn0, b0, Ws1, Wn1, b1, Ws2, Wn2, b2, edge_index):
    src = edge_index[0]
    dst = edge_index[1]
    zeros = jnp.zeros((DEG_PAD, D), jnp.float32)

    srcp = jnp.pad(src.reshape(NW, EPW), ((0, 0), (0, EPAD))).reshape(NW, NF, CH)
    trash = jnp.broadcast_to(N + (jnp.arange(EPAD, dtype=jnp.int32) % 8), (NW, EPAD))
    dstp = jnp.concatenate([dst.reshape(NW, EPW), trash], axis=1).reshape(NW, NF, CH)

    deg_raw = _sc_deg(dstp, zeros)
    deg = deg_raw[:, :N, 0:1]

    ws2p = jnp.zeros((D, D), jnp.float32).at[:, : Ws2.shape[1]].set(Ws2)
    wn2p = jnp.zeros((D, D), jnp.float32).at[:, : Wn2.shape[1]].set(Wn2)
    b2p = jnp.zeros((1, D), jnp.float32).at[0, : b2.shape[0]].set(b2)

    z0, y0 = _tc_pre(inputs, Ws0, Wn0, b0.reshape(1, D))
    acc0 = _sc_agg(y0, srcp, dstp, zeros)
    z1, y1 = _tc_postpre(z0, acc0, deg, Ws1, Wn1, b1.reshape(1, D))
    acc1 = _sc_agg(y1, srcp, dstp, zeros)
    z2, y2 = _tc_postpre(z1, acc1, deg, ws2p, wn2p, b2p)
    acc2 = _sc_agg(y2, srcp, dstp, zeros)
    out = _tc_post(z2, acc2, deg)
    return out[:, : Ws2.shape[1]]


# speedup vs baseline: 2.4063x; 2.4063x over previous
import functools

import jax
import jax.numpy as jnp
from jax import lax
from jax.experimental import pallas as pl
from jax.experimental.pallas import tpu as pltpu
from jax.experimental.pallas import tpu_sc as plsc

N = 10000
E = 320000
D = 128

NC = 2
NS = 16
NW = NC * NS
EPW = E // NW
CH = 128
NFULL = EPW // CH
REM = EPW - NFULL * CH

SLAB = 624
SLAB_TAIL = N - NS * SLAB

DEG_PER_TILE = 632
DEG_PAD = NS * DEG_PER_TILE

_MESH = plsc.VectorSubcoreMesh(core_axis_name="c", subcore_axis_name="s")


@functools.partial(
    pl.kernel,
    out_type=jax.ShapeDtypeStruct((NC, DEG_PAD, D), jnp.float32),
    mesh=_MESH,
    scratch_types=[
        pltpu.VMEM_SHARED((DEG_PAD, D), jnp.float32),
        pltpu.VMEM((CH,), jnp.int32),
        pltpu.VMEM((REM,), jnp.int32),
        pltpu.VMEM((CH, D), jnp.float32),
    ],
)
def _sc_deg(dst_hbm, zeros_hbm, out_hbm, acc_sh, dst_v, dst_r, ones_v):
    c = lax.axis_index("c")
    s = lax.axis_index("s")

    def fill_ones(i, _):
        r = i // 8
        k = i % 8
        ones_v[r, pl.ds(k * 16, 16)] = jnp.ones((16,), jnp.float32)
        return 0

    lax.fori_loop(0, CH * 8, fill_ones, 0)

    pltpu.sync_copy(
        zeros_hbm.at[pl.ds(s * DEG_PER_TILE, DEG_PER_TILE)],
        acc_sh.at[pl.ds(s * DEG_PER_TILE, DEG_PER_TILE)],
    )
    plsc.subcore_barrier()

    base = (c * NS + s) * EPW

    def body(j, _):
        pltpu.sync_copy(dst_hbm.at[pl.ds(base + j * CH, CH)], dst_v)
        pltpu.sync_copy(ones_v, acc_sh.at[dst_v], add=True)
        return 0

    lax.fori_loop(0, NFULL, body, 0)
    pltpu.sync_copy(dst_hbm.at[pl.ds(base + NFULL * CH, REM)], dst_r)
    pltpu.sync_copy(ones_v.at[pl.ds(0, REM)], acc_sh.at[dst_r], add=True)

    plsc.subcore_barrier()
    pltpu.sync_copy(
        acc_sh.at[pl.ds(s * DEG_PER_TILE, DEG_PER_TILE)],
        out_hbm.at[c, pl.ds(s * DEG_PER_TILE, DEG_PER_TILE)],
    )


NPAIR = NFULL // 2


@functools.partial(
    pl.kernel,
    out_type=jax.ShapeDtypeStruct((NC, N, D), jnp.float32),
    mesh=_MESH,
    scratch_types=[
        pltpu.VMEM_SHARED((N, D), jnp.float32),
    ] + [pltpu.VMEM((CH,), jnp.int32)] * 4
      + [pltpu.VMEM((CH, D), jnp.float32)] * 2 + [
        pltpu.VMEM((REM,), jnp.int32),
        pltpu.VMEM((REM,), jnp.int32),
        pltpu.VMEM((REM, D), jnp.float32),
    ] + [pltpu.SemaphoreType.DMA] * 2,
)
def _sc_agg(y_hbm, src_hbm, dst_hbm, zeros_hbm, out_hbm, acc_sh, *rest):
    src_a, dst_a, src_b, dst_b = rest[0:4]
    rows_a, rows_b = rest[4:6]
    src_r, dst_r, rows_r = rest[6:9]
    sem_a, sem_b = rest[9:11]
    c = lax.axis_index("c")
    s = lax.axis_index("s")

    pltpu.sync_copy(
        zeros_hbm.at[pl.ds(s * SLAB, SLAB)],
        acc_sh.at[pl.ds(s * SLAB, SLAB)],
    )

    @pl.when(s == 0)
    def _():
        pltpu.sync_copy(
            zeros_hbm.at[pl.ds(NS * SLAB, SLAB_TAIL)],
            acc_sh.at[pl.ds(NS * SLAB, SLAB_TAIL)],
        )

    plsc.subcore_barrier()

    base = (c * NS + s) * EPW

    pltpu.sync_copy(src_hbm.at[pl.ds(base, CH)], src_a)
    pltpu.sync_copy(dst_hbm.at[pl.ds(base, CH)], dst_a)
    pltpu.async_copy(y_hbm.at[src_a], rows_a, sem_a)
    pltpu.sync_copy(src_hbm.at[pl.ds(base + CH, CH)], src_b)
    pltpu.sync_copy(dst_hbm.at[pl.ds(base + CH, CH)], dst_b)
    pltpu.async_copy(y_hbm.at[src_b], rows_b, sem_b)

    def pair(t, _):
        pltpu.make_async_copy(y_hbm.at[pl.ds(0, CH)], rows_a, sem_a).wait()
        pltpu.sync_copy(rows_a, acc_sh.at[dst_a], add=True)

        @pl.when(t < NPAIR - 1)
        def _():
            off = base + (2 * t + 2) * CH
            pltpu.sync_copy(src_hbm.at[pl.ds(off, CH)], src_a)
            pltpu.sync_copy(dst_hbm.at[pl.ds(off, CH)], dst_a)
            pltpu.async_copy(y_hbm.at[src_a], rows_a, sem_a)

        pltpu.make_async_copy(y_hbm.at[pl.ds(0, CH)], rows_b, sem_b).wait()
        pltpu.sync_copy(rows_b, acc_sh.at[dst_b], add=True)

        @pl.when(t < NPAIR - 1)
        def _():
            off = base + (2 * t + 3) * CH
            pltpu.sync_copy(src_hbm.at[pl.ds(off, CH)], src_b)
            pltpu.sync_copy(dst_hbm.at[pl.ds(off, CH)], dst_b)
            pltpu.async_copy(y_hbm.at[src_b], rows_b, sem_b)

        return 0

    lax.fori_loop(0, NPAIR, pair, 0)

    off = base + NFULL * CH
    pltpu.sync_copy(src_hbm.at[pl.ds(off, REM)], src_r)
    pltpu.sync_copy(dst_hbm.at[pl.ds(off, REM)], dst_r)
    pltpu.async_copy(y_hbm.at[src_r], rows_r, sem_a).wait()
    pltpu.sync_copy(rows_r, acc_sh.at[dst_r], add=True)

    plsc.subcore_barrier()
    pltpu.sync_copy(
        acc_sh.at[pl.ds(s * SLAB, SLAB)],
        out_hbm.at[c, pl.ds(s * SLAB, SLAB)],
    )

    @pl.when(s == 0)
    def _():
        pltpu.sync_copy(
            acc_sh.at[pl.ds(NS * SLAB, SLAB_TAIL)],
            out_hbm.at[c, pl.ds(NS * SLAB, SLAB_TAIL)],
        )


_BLK = 2000


def _tc_pre_body(x_ref, ws_ref, wn_ref, b_ref, z_ref, y_ref):
    x = x_ref[...]
    z_ref[...] = jnp.dot(x, ws_ref[...], preferred_element_type=jnp.float32) + b_ref[...]
    y_ref[...] = jnp.dot(x, wn_ref[...], preferred_element_type=jnp.float32)


def _tc_pre(x, ws, wn, b):
    return pl.pallas_call(
        _tc_pre_body,
        grid=(N // _BLK,),
        in_specs=[
            pl.BlockSpec((_BLK, D), lambda i: (i, 0)),
            pl.BlockSpec((D, D), lambda i: (0, 0)),
            pl.BlockSpec((D, D), lambda i: (0, 0)),
            pl.BlockSpec((1, D), lambda i: (0, 0)),
        ],
        out_specs=[
            pl.BlockSpec((_BLK, D), lambda i: (i, 0)),
            pl.BlockSpec((_BLK, D), lambda i: (i, 0)),
        ],
        out_shape=[
            jax.ShapeDtypeStruct((N, D), jnp.float32),
            jax.ShapeDtypeStruct((N, D), jnp.float32),
        ],
    )(x, ws, wn, b)


def _combine(z_ref, acc_ref, deg_ref):
    d = deg_ref[0] + deg_ref[1]
    inv = 1.0 / jnp.maximum(d, 1.0)
    return z_ref[...] + (acc_ref[0] + acc_ref[1]) * inv


def _tc_postpre_body(z_ref, acc_ref, deg_ref, ws_ref, wn_ref, b_ref, z2_ref, y2_ref):
    h = jnp.maximum(_combine(z_ref, acc_ref, deg_ref), 0.0)
    z2_ref[...] = jnp.dot(h, ws_ref[...], preferred_element_type=jnp.float32) + b_ref[...]
    y2_ref[...] = jnp.dot(h, wn_ref[...], preferred_element_type=jnp.float32)


def _tc_postpre(z, acc, deg, ws, wn, b):
    return pl.pallas_call(
        _tc_postpre_body,
        grid=(N // _BLK,),
        in_specs=[
            pl.BlockSpec((_BLK, D), lambda i: (i, 0)),
            pl.BlockSpec((NC, _BLK, D), lambda i: (0, i, 0)),
            pl.BlockSpec((NC, _BLK, 1), lambda i: (0, i, 0)),
            pl.BlockSpec((D, D), lambda i: (0, 0)),
            pl.BlockSpec((D, D), lambda i: (0, 0)),
            pl.BlockSpec((1, D), lambda i: (0, 0)),
        ],
        out_specs=[
            pl.BlockSpec((_BLK, D), lambda i: (i, 0)),
            pl.BlockSpec((_BLK, D), lambda i: (i, 0)),
        ],
        out_shape=[
            jax.ShapeDtypeStruct((N, D), jnp.float32),
            jax.ShapeDtypeStruct((N, D), jnp.float32),
        ],
    )(z, acc, deg, ws, wn, b)


def _tc_post_body(z_ref, acc_ref, deg_ref, out_ref):
    out_ref[...] = _combine(z_ref, acc_ref, deg_ref)


def _tc_post(z, acc, deg):
    return pl.pallas_call(
        _tc_post_body,
        grid=(N // _BLK,),
        in_specs=[
            pl.BlockSpec((_BLK, D), lambda i: (i, 0)),
            pl.BlockSpec((NC, _BLK, D), lambda i: (0, i, 0)),
            pl.BlockSpec((NC, _BLK, 1), lambda i: (0, i, 0)),
        ],
        out_specs=pl.BlockSpec((_BLK, D), lambda i: (i, 0)),
        out_shape=jax.ShapeDtypeStruct((N, D), jnp.float32),
    )(z, acc, deg)


def kernel(inputs, Ws0, Wn0, b0, Ws1, Wn1, b1, Ws2, Wn2, b2, edge_index):
    src = edge_index[0]
    dst = edge_index[1]
    zeros = jnp.zeros((DEG_PAD, D), jnp.float32)

    deg_raw = _sc_deg(dst, zeros)
    deg = deg_raw[:, :N, 0:1]

    ws2p = jnp.zeros((D, D), jnp.float32).at[:, : Ws2.shape[1]].set(Ws2)
    wn2p = jnp.zeros((D, D), jnp.float32).at[:, : Wn2.shape[1]].set(Wn2)
    b2p = jnp.zeros((1, D), jnp.float32).at[0, : b2.shape[0]].set(b2)

    z0, y0 = _tc_pre(inputs, Ws0, Wn0, b0.reshape(1, D))
    acc0 = _sc_agg(y0, src, dst, zeros)
    z1, y1 = _tc_postpre(z0, acc0, deg, Ws1, Wn1, b1.reshape(1, D))
    acc1 = _sc_agg(y1, src, dst, zeros)
    z2, y2 = _tc_postpre(z1, acc1, deg, ws2p, wn2p, b2p)
    acc2 = _sc_agg(y2, src, dst, zeros)
    out = _tc_post(z2, acc2, deg)
    return out[:, : Ws2.shape[1]]
